# baseline (speedup 1.0000x reference)
import functools

import jax
import jax.numpy as jnp
from jax import lax
from jax.experimental import pallas as pl
from jax.experimental.pallas import tpu as pltpu
from jax.experimental.pallas import tpu_sc as plsc

_NUM_EXPERTS = 16
_L = 16
_NS = 16
_TOKENS = 4 * 4096
_TOK_PER_TILE = _TOKENS // _NS
_CHUNKS = _TOK_PER_TILE // _L

_TOTAL_CAPITAL = 10000.0
_MIN_CAP = _TOTAL_CAPITAL * 0.05 / _NUM_EXPERTS
_TAX_THRESHOLD = 2.0
_TAX_RATE = 0.1


def _sc_body(loss_h, cost_h, w0_h, w1_h, caps_h, base_h, out_h,
             loss_v, cost_v, w0_v, w1_v, cnt_v, s_v, part_v,
             caps_v, base_v, shared, all_v, res_v):
    sid = lax.axis_index("s")
    off = sid * _TOK_PER_TILE
    pltpu.sync_copy(loss_h.at[pl.ds(off, _TOK_PER_TILE)], loss_v)
    pltpu.sync_copy(cost_h.at[pl.ds(off, _TOK_PER_TILE)], cost_v)
    pltpu.sync_copy(w0_h.at[pl.ds(off, _TOK_PER_TILE)], w0_v)
    pltpu.sync_copy(w1_h.at[pl.ds(off, _TOK_PER_TILE)], w1_v)

    zeros = jnp.zeros((_L,), jnp.float32)
    ones = jnp.ones((_L,), jnp.float32)
    cnt_v[...] = zeros
    s_v[...] = zeros
    loss_acc = zeros
    for i in range(_CHUNKS):
        sl = pl.ds(i * _L, _L)
        lo = loss_v[sl]
        lc = lo + cost_v[sl]
        a = w0_v[sl]
        b = w1_v[sl]
        m = b != a
        plsc.addupdate_scatter(s_v, [a], lc)
        plsc.addupdate_scatter(s_v, [b], lc, mask=m)
        plsc.addupdate_scatter(cnt_v, [a], ones)
        plsc.addupdate_scatter(cnt_v, [b], ones, mask=m)
        loss_acc = loss_acc + lo

    part_v[pl.ds(0, _L)] = cnt_v[...]
    part_v[pl.ds(_L, _L)] = s_v[...]
    part_v[pl.ds(2 * _L, _L)] = loss_acc
    pltpu.sync_copy(part_v, shared.at[pl.ds(sid * 3 * _L, 3 * _L)])
    plsc.subcore_barrier()

    @pl.when(sid == 0)
    def _finalize():
        pltpu.sync_copy(shared, all_v)
        pltpu.sync_copy(caps_h, caps_v)
        pltpu.sync_copy(base_h, base_v)
        cnt = all_v[pl.ds(0, _L)]
        s = all_v[pl.ds(_L, _L)]
        lsum = all_v[pl.ds(2 * _L, _L)]
        for i in range(1, _NS):
            cnt = cnt + all_v[pl.ds(i * 3 * _L, _L)]
            s = s + all_v[pl.ds((i * 3 + 1) * _L, _L)]
            lsum = lsum + all_v[pl.ds((i * 3 + 2) * _L, _L)]
        avg_loss = jnp.sum(lsum) * (1.0 / _TOKENS)
        new_base = 0.99 * base_v[...] + 0.01 * avg_loss
        caps = caps_v[...] + new_base * cnt - s
        thr = jnp.sum(caps) * (_TAX_THRESHOLD / _NUM_EXPERTS)
        caps = jnp.where(caps > thr, caps - (caps - thr) * _TAX_RATE, caps)
        caps = jnp.maximum(caps, _MIN_CAP)
        total = jnp.sum(caps)
        scale = jnp.where(total > _TOTAL_CAPITAL * 1.5, 0.95, 1.0)
        shift = jnp.where(total < _TOTAL_CAPITAL * 0.5, _TOTAL_CAPITAL * 0.01, 0.0)
        caps = caps * scale + shift
        res_v[pl.ds(0, _L)] = caps
        res_v[pl.ds(_L, _L)] = new_base
        pltpu.sync_copy(res_v, out_h)


_mesh = plsc.VectorSubcoreMesh(
    core_axis_name="c", subcore_axis_name="s", num_cores=1, num_subcores=_NS)

_sc_call = pl.kernel(
    _sc_body,
    out_type=jax.ShapeDtypeStruct((2 * _L,), jnp.float32),
    mesh=_mesh,
    scratch_types=[
        pltpu.VMEM((_TOK_PER_TILE,), jnp.float32),
        pltpu.VMEM((_TOK_PER_TILE,), jnp.float32),
        pltpu.VMEM((_TOK_PER_TILE,), jnp.int32),
        pltpu.VMEM((_TOK_PER_TILE,), jnp.int32),
        pltpu.VMEM((_NUM_EXPERTS,), jnp.float32),
        pltpu.VMEM((_NUM_EXPERTS,), jnp.float32),
        pltpu.VMEM((3 * _L,), jnp.float32),
        pltpu.VMEM((_L,), jnp.float32),
        pltpu.VMEM((_L,), jnp.float32),
        pltpu.VMEM_SHARED((_NS * 3 * _L,), jnp.float32),
        pltpu.VMEM((_NS * 3 * _L,), jnp.float32),
        pltpu.VMEM((2 * _L,), jnp.float32),
    ],
    compiler_params=pltpu.CompilerParams(needs_layout_passes=False),
    name="capital_manager_sc",
)


def kernel(capitals, baseline_losses, token_losses, costs, winners, layer_idx):
    li = jnp.asarray(layer_idx, jnp.int32)
    caps_row = lax.dynamic_index_in_dim(capitals, li, axis=0, keepdims=False)
    base_val = lax.dynamic_index_in_dim(baseline_losses, li, axis=0,
                                        keepdims=False)
    loss_flat = token_losses.reshape(_TOKENS)
    cost_flat = costs.reshape(_TOKENS)
    w = winners.reshape(_TOKENS, 2)
    w0 = w[:, 0]
    w1 = w[:, 1]
    base_vec = jnp.full((_L,), base_val, dtype=jnp.float32)
    out = _sc_call(loss_flat, cost_flat, w0, w1, caps_row, base_vec)
    new_caps_row = out[:_L]
    new_base = out[_L]
    new_capitals = capitals.at[li].set(new_caps_row)
    new_baselines = baseline_losses.at[li].set(new_base)
    return new_capitals, new_baselines
